# baseline (speedup 1.0000x reference)
import functools

import jax
import jax.numpy as jnp
from jax import lax
from jax.experimental import pallas as pl
from jax.experimental.pallas import tpu as pltpu
from jax.experimental.pallas import tpu_sc as plsc

_N = 320000
_F = 128
_S = 512
_EPS = 0.001

_B = 6400
_NB = _N // _B
_W = 32
_SPAD = 640

_NC = 2
_NS = 16
_NWORK = _NC * _NS
_RPW = _N // _NWORK
_C = 400
_NCH = _RPW // _C
_SUB = 80
_NSUB = _C // _SUB
_TROWS = _SPAD // _NS


def _sc_stats(x, seg):
    mesh = plsc.VectorSubcoreMesh(core_axis_name="c", subcore_axis_name="s")
    kernel_fn = pl.kernel(
        _sc_stats_kernel,
        out_type=[
            jax.ShapeDtypeStruct((_NC, _SPAD, _F), jnp.float32),
            jax.ShapeDtypeStruct((_NC, _SPAD, _F), jnp.float32),
            jax.ShapeDtypeStruct((_NC, _SPAD, _F), jnp.float32),
        ],
        mesh=mesh,
        scratch_types=[
            pltpu.VMEM((_C, _F), jnp.float32),
            pltpu.VMEM((_SUB,), jnp.int32),
            pltpu.VMEM((_SUB, _F), jnp.float32),
            pltpu.VMEM((_TROWS, _F), jnp.float32),
            pltpu.VMEM_SHARED((_SPAD, _F), jnp.float32),
            pltpu.VMEM_SHARED((_SPAD, _F), jnp.float32),
            pltpu.VMEM_SHARED((_SPAD, _F), jnp.float32),
        ],
    )
    return kernel_fn(x, seg)


def _sc_stats_kernel(x_hbm, seg_hbm, sum_hbm, sq_hbm, cnt_hbm,
                     xbuf, idxbuf, onesbuf, zbuf,
                     sum_sh, sq_sh, cnt_sh):
    cid = lax.axis_index("c")
    sid = lax.axis_index("s")
    wid = cid * _NS + sid
    row0 = wid * _RPW

    @pl.loop(0, _TROWS)
    def _zrow(r):
        for f in range(_F // 16):
            zbuf[r, pl.ds(f * 16, 16)] = jnp.zeros((16,), jnp.float32)

    @pl.loop(0, _SUB)
    def _orow(r):
        for f in range(_F // 16):
            onesbuf[r, pl.ds(f * 16, 16)] = jnp.ones((16,), jnp.float32)

    t0 = sid * _TROWS
    pltpu.sync_copy(zbuf, sum_sh.at[pl.ds(t0, _TROWS)])
    pltpu.sync_copy(zbuf, sq_sh.at[pl.ds(t0, _TROWS)])
    pltpu.sync_copy(zbuf, cnt_sh.at[pl.ds(t0, _TROWS)])
    plsc.subcore_barrier()

    @pl.loop(0, _NCH)
    def _chunk(k):
        base = row0 + k * _C
        pltpu.sync_copy(x_hbm.at[pl.ds(base, _C)], xbuf)

        for j in range(_NSUB):
            pltpu.sync_copy(seg_hbm.at[pl.ds(base + j * _SUB, _SUB)], idxbuf)
            pltpu.sync_copy(xbuf.at[pl.ds(j * _SUB, _SUB)],
                            sum_sh.at[idxbuf], add=True)
            pltpu.sync_copy(onesbuf, cnt_sh.at[idxbuf], add=True)

        @pl.loop(0, _C)
        def _sqrow(r):
            for f in range(_F // 16):
                v = xbuf[r, pl.ds(f * 16, 16)]
                xbuf[r, pl.ds(f * 16, 16)] = v * v

        for j in range(_NSUB):
            pltpu.sync_copy(seg_hbm.at[pl.ds(base + j * _SUB, _SUB)], idxbuf)
            pltpu.sync_copy(xbuf.at[pl.ds(j * _SUB, _SUB)],
                            sq_sh.at[idxbuf], add=True)

    plsc.subcore_barrier()

    pltpu.sync_copy(sum_sh.at[pl.ds(t0, _TROWS)],
                    sum_hbm.at[cid, pl.ds(t0, _TROWS)])
    pltpu.sync_copy(sq_sh.at[pl.ds(t0, _TROWS)],
                    sq_hbm.at[cid, pl.ds(t0, _TROWS)])
    pltpu.sync_copy(cnt_sh.at[pl.ds(t0, _TROWS)],
                    cnt_hbm.at[cid, pl.ds(t0, _TROWS)])


def _finalize_kernel(sum_ref, sq_ref, cnt_ref, table_ref):
    sums = sum_ref[0] + sum_ref[1]
    sqs = sq_ref[0] + sq_ref[1]
    cnt = (cnt_ref[0] + cnt_ref[1])[:, 0:1]
    mean = sums / jnp.maximum(cnt, 1.0)
    ssq = (jnp.sum(sqs, axis=1, keepdims=True)
           - cnt * jnp.sum(mean * mean, axis=1, keepdims=True))
    var = ssq / (cnt * jnp.float32(_F) - 1.0)
    invstd = lax.rsqrt(var + _EPS)
    table_ref[:, 0:_F] = mean.astype(jnp.bfloat16)
    table_ref[:, _F:2 * _F] = jnp.broadcast_to(
        invstd, (_SPAD, _F)).astype(jnp.bfloat16)


def _norm_kernel(x_ref, seg_ref, table_ref, out_ref):
    seg = seg_ref[0, 0, :]
    s0 = (seg[0] // 16) * 16
    smax = seg[_B - 1]
    x = x_ref[...]

    narrow = (smax - s0) < _W

    @pl.when(narrow)
    def _narrow():
        win = table_ref[pl.ds(s0, _W), :]
        col = lax.broadcasted_iota(jnp.int32, (_B, _W), 1)
        oh = (seg[:, None] - s0 == col).astype(
            jnp.float32).astype(jnp.bfloat16)
        rows = lax.dot_general(
            oh, win, (((1,), (0,)), ((), ())),
            preferred_element_type=jnp.float32)
        out_ref[...] = (x - rows[:, 0:_F]) * rows[:, _F:2 * _F]

    @pl.when(jnp.logical_not(narrow))
    def _wide():
        win = table_ref[pl.ds(0, _S), :]
        col = lax.broadcasted_iota(jnp.int32, (_B, _S), 1)
        oh = (seg[:, None] == col).astype(
            jnp.float32).astype(jnp.bfloat16)
        rows = lax.dot_general(
            oh, win, (((1,), (0,)), ((), ())),
            preferred_element_type=jnp.float32)
        out_ref[...] = (x - rows[:, 0:_F]) * rows[:, _F:2 * _F]


def kernel(x, i):
    seg = i.astype(jnp.int32)
    seg3 = seg.reshape(_NB, 1, _B)

    sums, sqs, cnts = _sc_stats(x, seg)

    table = pl.pallas_call(
        _finalize_kernel,
        in_specs=[
            pl.BlockSpec((_NC, _SPAD, _F), lambda: (0, 0, 0)),
            pl.BlockSpec((_NC, _SPAD, _F), lambda: (0, 0, 0)),
            pl.BlockSpec((_NC, _SPAD, _F), lambda: (0, 0, 0)),
        ],
        out_specs=pl.BlockSpec((_SPAD, 2 * _F), lambda: (0, 0)),
        out_shape=jax.ShapeDtypeStruct((_SPAD, 2 * _F), jnp.bfloat16),
    )(sums, sqs, cnts)

    out = pl.pallas_call(
        _norm_kernel,
        grid=(_NB,),
        in_specs=[
            pl.BlockSpec((_B, _F), lambda b: (b, 0)),
            pl.BlockSpec((1, 1, _B), lambda b: (b, 0, 0)),
            pl.BlockSpec((_SPAD, 2 * _F), lambda b: (0, 0)),
        ],
        out_specs=pl.BlockSpec((_B, _F), lambda b: (b, 0)),
        out_shape=jax.ShapeDtypeStruct((_N, _F), jnp.float32),
    )(x, seg3, table)

    return out


# speedup vs baseline: 2.8000x; 2.8000x over previous
import jax
import jax.numpy as jnp
from jax.experimental import pallas as pl
from jax.experimental.pallas import tpu as pltpu

_N = 320000
_F = 128
_S = 512
_EPS = 0.001
_B = 6400
_NB = _N // _B
_W = 32
_SPAD = _S + _W


def _accumulate(oh_bf, oh_f32, x_bf, s0, w, acc_sum, acc_sq, acc_cnt):
    dims = (((0,), (0,)), ((), ()))
    ps = jax.lax.dot_general(oh_bf, x_bf, dims,
                             preferred_element_type=jnp.float32)
    psq = jax.lax.dot_general(oh_bf, x_bf * x_bf, dims,
                              preferred_element_type=jnp.float32)
    cnt = jnp.sum(oh_f32, axis=0).reshape(w, 1)
    acc_sum[pl.ds(s0, w), :] += ps
    acc_sq[pl.ds(s0, w), :] += psq
    acc_cnt[pl.ds(s0, w), :] += cnt


def _stats_kernel(x_ref, seg_ref, table_ref, acc_sum, acc_sq, acc_cnt):
    b = pl.program_id(0)

    @pl.when(b == 0)
    def _init():
        acc_sum[...] = jnp.zeros_like(acc_sum)
        acc_sq[...] = jnp.zeros_like(acc_sq)
        acc_cnt[...] = jnp.zeros_like(acc_cnt)

    seg = seg_ref[0, 0, :]
    s0 = (seg[0] // 8) * 8
    smax = seg[_B - 1]
    x_bf = x_ref[...].astype(jnp.bfloat16)

    narrow = (smax - s0) < _W

    @pl.when(narrow)
    def _narrow():
        col = jax.lax.broadcasted_iota(jnp.int32, (_B, _W), 1)
        oh_f32 = (seg[:, None] - s0 == col).astype(jnp.float32)
        _accumulate(oh_f32.astype(jnp.bfloat16), oh_f32, x_bf,
                    s0, _W, acc_sum, acc_sq, acc_cnt)

    @pl.when(jnp.logical_not(narrow))
    def _wide():
        col = jax.lax.broadcasted_iota(jnp.int32, (_B, _S), 1)
        oh_f32 = (seg[:, None] == col).astype(jnp.float32)
        _accumulate(oh_f32.astype(jnp.bfloat16), oh_f32, x_bf,
                    0, _S, acc_sum, acc_sq, acc_cnt)

    @pl.when(b == _NB - 1)
    def _finalize():
        cnt = acc_cnt[...]
        mean = acc_sum[...] / jnp.maximum(cnt, 1.0)
        ssq = (jnp.sum(acc_sq[...], axis=1, keepdims=True)
               - cnt * jnp.sum(mean * mean, axis=1, keepdims=True))
        var = ssq / (cnt * jnp.float32(_F) - 1.0)
        invstd = jax.lax.rsqrt(var + _EPS)
        table_ref[:, 0:_F] = (mean * invstd).astype(jnp.bfloat16)
        table_ref[:, _F:2 * _F] = jnp.broadcast_to(
            invstd, (_SPAD, _F)).astype(jnp.bfloat16)


def _norm_kernel(x_ref, seg_ref, table_ref, out_ref):
    seg = seg_ref[0, 0, :]
    s0 = (seg[0] // 16) * 16
    smax = seg[_B - 1]
    x = x_ref[...]

    narrow = (smax - s0) < _W

    @pl.when(narrow)
    def _narrow():
        win = table_ref[pl.ds(s0, _W), :]
        col = jax.lax.broadcasted_iota(jnp.int32, (_B, _W), 1)
        oh = (seg[:, None] - s0 == col).astype(
            jnp.float32).astype(jnp.bfloat16)
        rows = jax.lax.dot_general(
            oh, win, (((1,), (0,)), ((), ())),
            preferred_element_type=jnp.float32)
        out_ref[...] = x * rows[:, _F:2 * _F] - rows[:, 0:_F]

    @pl.when(jnp.logical_not(narrow))
    def _wide():
        win = table_ref[pl.ds(0, _S), :]
        col = jax.lax.broadcasted_iota(jnp.int32, (_B, _S), 1)
        oh = (seg[:, None] == col).astype(
            jnp.float32).astype(jnp.bfloat16)
        rows = jax.lax.dot_general(
            oh, win, (((1,), (0,)), ((), ())),
            preferred_element_type=jnp.float32)
        out_ref[...] = x * rows[:, _F:2 * _F] - rows[:, 0:_F]


def kernel(x, i):
    seg = i.astype(jnp.int32)
    seg3 = seg.reshape(_NB, 1, _B)

    table = pl.pallas_call(
        _stats_kernel,
        grid=(_NB,),
        in_specs=[
            pl.BlockSpec((_B, _F), lambda b: (b, 0)),
            pl.BlockSpec((1, 1, _B), lambda b: (b, 0, 0)),
        ],
        out_specs=pl.BlockSpec((_SPAD, 2 * _F), lambda b: (0, 0)),
        out_shape=jax.ShapeDtypeStruct((_SPAD, 2 * _F), jnp.bfloat16),
        scratch_shapes=[
            pltpu.VMEM((_SPAD, _F), jnp.float32),
            pltpu.VMEM((_SPAD, _F), jnp.float32),
            pltpu.VMEM((_SPAD, 1), jnp.float32),
        ],
    )(x, seg3)

    out = pl.pallas_call(
        _norm_kernel,
        grid=(_NB,),
        in_specs=[
            pl.BlockSpec((_B, _F), lambda b: (b, 0)),
            pl.BlockSpec((1, 1, _B), lambda b: (b, 0, 0)),
            pl.BlockSpec((_SPAD, 2 * _F), lambda b: (0, 0)),
        ],
        out_specs=pl.BlockSpec((_B, _F), lambda b: (b, 0)),
        out_shape=jax.ShapeDtypeStruct((_N, _F), jnp.float32),
    )(x, seg3, table)

    return out


# speedup vs baseline: 2.9412x; 1.0504x over previous
import jax
import jax.numpy as jnp
from jax.experimental import pallas as pl
from jax.experimental.pallas import tpu as pltpu

_N = 320000
_F = 128
_S = 512
_EPS = 0.001
_B = 8000
_NB = _N // _B
_W = 32
_SPAD = _S + _W


def _accumulate(oh_bf, oh_f32, x_bf, s0, w, acc_sum, acc_sq, acc_cnt):
    dims = (((0,), (0,)), ((), ()))
    ps = jax.lax.dot_general(oh_bf, x_bf, dims,
                             preferred_element_type=jnp.float32)
    psq = jax.lax.dot_general(oh_bf, x_bf * x_bf, dims,
                              preferred_element_type=jnp.float32)
    cnt = jnp.sum(oh_f32, axis=0).reshape(w, 1)
    acc_sum[pl.ds(s0, w), :] += ps
    acc_sq[pl.ds(s0, w), :] += psq
    acc_cnt[pl.ds(s0, w), :] += cnt


def _stats_kernel(x_ref, seg_ref, table_ref, acc_sum, acc_sq, acc_cnt):
    b = pl.program_id(0)

    @pl.when(b == 0)
    def _init():
        acc_sum[...] = jnp.zeros_like(acc_sum)
        acc_sq[...] = jnp.zeros_like(acc_sq)
        acc_cnt[...] = jnp.zeros_like(acc_cnt)

    seg = seg_ref[0, 0, :]
    s0 = (seg[0] // 8) * 8
    smax = seg[_B - 1]
    x_bf = x_ref[...].astype(jnp.bfloat16)

    narrow = (smax - s0) < _W

    @pl.when(narrow)
    def _narrow():
        col = jax.lax.broadcasted_iota(jnp.int32, (_B, _W), 1)
        oh_f32 = (seg[:, None] - s0 == col).astype(jnp.float32)
        _accumulate(oh_f32.astype(jnp.bfloat16), oh_f32, x_bf,
                    s0, _W, acc_sum, acc_sq, acc_cnt)

    @pl.when(jnp.logical_not(narrow))
    def _wide():
        col = jax.lax.broadcasted_iota(jnp.int32, (_B, _S), 1)
        oh_f32 = (seg[:, None] == col).astype(jnp.float32)
        _accumulate(oh_f32.astype(jnp.bfloat16), oh_f32, x_bf,
                    0, _S, acc_sum, acc_sq, acc_cnt)

    @pl.when(b == _NB - 1)
    def _finalize():
        cnt = acc_cnt[...]
        mean = acc_sum[...] / jnp.maximum(cnt, 1.0)
        ssq = (jnp.sum(acc_sq[...], axis=1, keepdims=True)
               - cnt * jnp.sum(mean * mean, axis=1, keepdims=True))
        var = ssq / (cnt * jnp.float32(_F) - 1.0)
        invstd = jax.lax.rsqrt(var + _EPS)
        table_ref[:, 0:_F] = (mean * invstd).astype(jnp.bfloat16)
        table_ref[:, _F:2 * _F] = jnp.broadcast_to(
            invstd, (_SPAD, _F)).astype(jnp.bfloat16)


def _norm_kernel(x_ref, seg_ref, table_ref, out_ref):
    seg = seg_ref[0, 0, :]
    s0 = (seg[0] // 16) * 16
    smax = seg[_B - 1]
    x = x_ref[...]

    narrow = (smax - s0) < _W

    @pl.when(narrow)
    def _narrow():
        win = table_ref[pl.ds(s0, _W), :]
        col = jax.lax.broadcasted_iota(jnp.int32, (_B, _W), 1)
        oh = (seg[:, None] - s0 == col).astype(
            jnp.float32).astype(jnp.bfloat16)
        rows = jax.lax.dot_general(
            oh, win, (((1,), (0,)), ((), ())),
            preferred_element_type=jnp.float32)
        out_ref[...] = x * rows[:, _F:2 * _F] - rows[:, 0:_F]

    @pl.when(jnp.logical_not(narrow))
    def _wide():
        win = table_ref[pl.ds(0, _S), :]
        col = jax.lax.broadcasted_iota(jnp.int32, (_B, _S), 1)
        oh = (seg[:, None] == col).astype(
            jnp.float32).astype(jnp.bfloat16)
        rows = jax.lax.dot_general(
            oh, win, (((1,), (0,)), ((), ())),
            preferred_element_type=jnp.float32)
        out_ref[...] = x * rows[:, _F:2 * _F] - rows[:, 0:_F]


def kernel(x, i):
    seg = i.astype(jnp.int32)
    seg3 = seg.reshape(_NB, 1, _B)

    table = pl.pallas_call(
        _stats_kernel,
        grid=(_NB,),
        in_specs=[
            pl.BlockSpec((_B, _F), lambda b: (b, 0)),
            pl.BlockSpec((1, 1, _B), lambda b: (b, 0, 0)),
        ],
        out_specs=pl.BlockSpec((_SPAD, 2 * _F), lambda b: (0, 0)),
        out_shape=jax.ShapeDtypeStruct((_SPAD, 2 * _F), jnp.bfloat16),
        scratch_shapes=[
            pltpu.VMEM((_SPAD, _F), jnp.float32),
            pltpu.VMEM((_SPAD, _F), jnp.float32),
            pltpu.VMEM((_SPAD, 1), jnp.float32),
        ],
    )(x, seg3)

    out = pl.pallas_call(
        _norm_kernel,
        grid=(_NB,),
        in_specs=[
            pl.BlockSpec((_B, _F), lambda b: (b, 0)),
            pl.BlockSpec((1, 1, _B), lambda b: (b, 0, 0)),
            pl.BlockSpec((_SPAD, 2 * _F), lambda b: (0, 0)),
        ],
        out_specs=pl.BlockSpec((_B, _F), lambda b: (b, 0)),
        out_shape=jax.ShapeDtypeStruct((_N, _F), jnp.float32),
    )(x, seg3, table)

    return out


# speedup vs baseline: 2.9953x; 1.0184x over previous
import jax
import jax.numpy as jnp
from jax.experimental import pallas as pl
from jax.experimental.pallas import tpu as pltpu

_N = 320000
_F = 128
_S = 512
_EPS = 0.001
_B = 10000
_NB = _N // _B
_W = 32
_SPAD = _S + _W


def _accumulate(oh_bf, oh_f32, x_bf, s0, w, acc_sum, acc_sq, acc_cnt):
    dims = (((0,), (0,)), ((), ()))
    ps = jax.lax.dot_general(oh_bf, x_bf, dims,
                             preferred_element_type=jnp.float32)
    psq = jax.lax.dot_general(oh_bf, x_bf * x_bf, dims,
                              preferred_element_type=jnp.float32)
    cnt = jnp.sum(oh_f32, axis=0).reshape(w, 1)
    acc_sum[pl.ds(s0, w), :] += ps
    acc_sq[pl.ds(s0, w), :] += psq
    acc_cnt[pl.ds(s0, w), :] += cnt


def _stats_kernel(x_ref, seg_ref, table_ref, acc_sum, acc_sq, acc_cnt):
    b = pl.program_id(0)

    @pl.when(b == 0)
    def _init():
        acc_sum[...] = jnp.zeros_like(acc_sum)
        acc_sq[...] = jnp.zeros_like(acc_sq)
        acc_cnt[...] = jnp.zeros_like(acc_cnt)

    seg = seg_ref[0, 0, :]
    s0 = (seg[0] // 8) * 8
    smax = seg[_B - 1]
    x_bf = x_ref[...].astype(jnp.bfloat16)

    narrow = (smax - s0) < _W

    @pl.when(narrow)
    def _narrow():
        col = jax.lax.broadcasted_iota(jnp.int32, (_B, _W), 1)
        oh_f32 = (seg[:, None] - s0 == col).astype(jnp.float32)
        _accumulate(oh_f32.astype(jnp.bfloat16), oh_f32, x_bf,
                    s0, _W, acc_sum, acc_sq, acc_cnt)

    @pl.when(jnp.logical_not(narrow))
    def _wide():
        col = jax.lax.broadcasted_iota(jnp.int32, (_B, _S), 1)
        oh_f32 = (seg[:, None] == col).astype(jnp.float32)
        _accumulate(oh_f32.astype(jnp.bfloat16), oh_f32, x_bf,
                    0, _S, acc_sum, acc_sq, acc_cnt)

    @pl.when(b == _NB - 1)
    def _finalize():
        cnt = acc_cnt[...]
        mean = acc_sum[...] / jnp.maximum(cnt, 1.0)
        ssq = (jnp.sum(acc_sq[...], axis=1, keepdims=True)
               - cnt * jnp.sum(mean * mean, axis=1, keepdims=True))
        var = ssq / (cnt * jnp.float32(_F) - 1.0)
        invstd = jax.lax.rsqrt(var + _EPS)
        table_ref[:, 0:_F] = (mean * invstd).astype(jnp.bfloat16)
        table_ref[:, _F:2 * _F] = jnp.broadcast_to(
            invstd, (_SPAD, _F)).astype(jnp.bfloat16)


def _norm_kernel(x_ref, seg_ref, table_ref, out_ref):
    seg = seg_ref[0, 0, :]
    s0 = (seg[0] // 16) * 16
    smax = seg[_B - 1]
    x = x_ref[...]

    narrow = (smax - s0) < _W

    @pl.when(narrow)
    def _narrow():
        win = table_ref[pl.ds(s0, _W), :]
        col = jax.lax.broadcasted_iota(jnp.int32, (_B, _W), 1)
        oh = (seg[:, None] - s0 == col).astype(
            jnp.float32).astype(jnp.bfloat16)
        rows = jax.lax.dot_general(
            oh, win, (((1,), (0,)), ((), ())),
            preferred_element_type=jnp.float32)
        out_ref[...] = x * rows[:, _F:2 * _F] - rows[:, 0:_F]

    @pl.when(jnp.logical_not(narrow))
    def _wide():
        win = table_ref[pl.ds(0, _S), :]
        col = jax.lax.broadcasted_iota(jnp.int32, (_B, _S), 1)
        oh = (seg[:, None] == col).astype(
            jnp.float32).astype(jnp.bfloat16)
        rows = jax.lax.dot_general(
            oh, win, (((1,), (0,)), ((), ())),
            preferred_element_type=jnp.float32)
        out_ref[...] = x * rows[:, _F:2 * _F] - rows[:, 0:_F]


def kernel(x, i):
    seg = i.astype(jnp.int32)
    seg3 = seg.reshape(_NB, 1, _B)

    table = pl.pallas_call(
        _stats_kernel,
        grid=(_NB,),
        in_specs=[
            pl.BlockSpec((_B, _F), lambda b: (b, 0)),
            pl.BlockSpec((1, 1, _B), lambda b: (b, 0, 0)),
        ],
        out_specs=pl.BlockSpec((_SPAD, 2 * _F), lambda b: (0, 0)),
        out_shape=jax.ShapeDtypeStruct((_SPAD, 2 * _F), jnp.bfloat16),
        scratch_shapes=[
            pltpu.VMEM((_SPAD, _F), jnp.float32),
            pltpu.VMEM((_SPAD, _F), jnp.float32),
            pltpu.VMEM((_SPAD, 1), jnp.float32),
        ],
    )(x, seg3)

    out = pl.pallas_call(
        _norm_kernel,
        grid=(_NB,),
        in_specs=[
            pl.BlockSpec((_B, _F), lambda b: (b, 0)),
            pl.BlockSpec((1, 1, _B), lambda b: (b, 0, 0)),
            pl.BlockSpec((_SPAD, 2 * _F), lambda b: (0, 0)),
        ],
        out_specs=pl.BlockSpec((_B, _F), lambda b: (b, 0)),
        out_shape=jax.ShapeDtypeStruct((_N, _F), jnp.float32),
    )(x, seg3, table)

    return out


# speedup vs baseline: 3.0947x; 1.0332x over previous
import jax
import jax.numpy as jnp
from jax import lax
from jax.experimental import pallas as pl
from jax.experimental.pallas import tpu as pltpu

_N = 320000
_F = 128
_S = 512
_EPS = 0.001
_B = 10000
_NB = _N // _B
_W = 32
_SPAD = _S + 2 * _W


def _accumulate(oh_bf, oh_f32, x_bf, s0, w, acc_sum, acc_sq, acc_cnt):
    dims = (((0,), (0,)), ((), ()))
    ps = lax.dot_general(oh_bf, x_bf, dims,
                         preferred_element_type=jnp.float32)
    psq = lax.dot_general(oh_bf, x_bf * x_bf, dims,
                          preferred_element_type=jnp.float32)
    cnt = jnp.sum(oh_f32, axis=0).reshape(w, 1)
    acc_sum[pl.ds(s0, w), :] += ps
    acc_sq[pl.ds(s0, w), :] += psq
    acc_cnt[pl.ds(s0, w), :] += cnt


def _fused_kernel(x_ref, seg_ref, out_ref, acc_sum, acc_sq, acc_cnt, table):
    g = pl.program_id(0)

    @pl.when(g == 0)
    def _init():
        acc_sum[...] = jnp.zeros_like(acc_sum)
        acc_sq[...] = jnp.zeros_like(acc_sq)
        acc_cnt[...] = jnp.zeros_like(acc_cnt)

    seg = seg_ref[0, 0, :]
    smax = seg[_B - 1]

    @pl.when(g < _NB)
    def _stats_phase():
        s0 = (seg[0] // 8) * 8
        narrow = (smax - s0) < _W

        @pl.when(narrow)
        def _narrow():
            col = lax.broadcasted_iota(jnp.int32, (_B, _W), 1)
            oh_f32 = (seg[:, None] - s0 == col).astype(jnp.float32)
            _accumulate(oh_f32.astype(jnp.bfloat16), oh_f32,
                        x_ref[...].astype(jnp.bfloat16),
                        s0, _W, acc_sum, acc_sq, acc_cnt)

        @pl.when(jnp.logical_not(narrow))
        def _wide():
            col = lax.broadcasted_iota(jnp.int32, (_B, _S), 1)
            oh_f32 = (seg[:, None] == col).astype(jnp.float32)
            _accumulate(oh_f32.astype(jnp.bfloat16), oh_f32,
                        x_ref[...].astype(jnp.bfloat16),
                        0, _S, acc_sum, acc_sq, acc_cnt)

        @pl.when(g == _NB - 1)
        def _finalize():
            cnt = acc_cnt[...]
            mean = acc_sum[...] / jnp.maximum(cnt, 1.0)
            ssq = (jnp.sum(acc_sq[...], axis=1, keepdims=True)
                   - cnt * jnp.sum(mean * mean, axis=1, keepdims=True))
            var = ssq / (cnt * jnp.float32(_F) - 1.0)
            invstd = lax.rsqrt(var + _EPS)
            table[:, 0:_F] = (mean * invstd).astype(jnp.bfloat16)
            table[:, _F:2 * _F] = jnp.broadcast_to(
                invstd, (_SPAD, _F)).astype(jnp.bfloat16)

    @pl.when(g >= _NB)
    def _norm_phase():
        s0 = (seg[0] // 16) * 16
        x = x_ref[...]
        narrow = (smax - s0) < _W

        @pl.when(narrow)
        def _narrow():
            win = table[pl.ds(s0, _W), :]
            col = lax.broadcasted_iota(jnp.int32, (_B, _W), 1)
            oh = (seg[:, None] - s0 == col).astype(
                jnp.float32).astype(jnp.bfloat16)
            rows = lax.dot_general(
                oh, win, (((1,), (0,)), ((), ())),
                preferred_element_type=jnp.float32)
            out_ref[...] = x * rows[:, _F:2 * _F] - rows[:, 0:_F]

        @pl.when(jnp.logical_not(narrow))
        def _wide():
            win = table[pl.ds(0, _S), :]
            col = lax.broadcasted_iota(jnp.int32, (_B, _S), 1)
            oh = (seg[:, None] == col).astype(
                jnp.float32).astype(jnp.bfloat16)
            rows = lax.dot_general(
                oh, win, (((1,), (0,)), ((), ())),
                preferred_element_type=jnp.float32)
            out_ref[...] = x * rows[:, _F:2 * _F] - rows[:, 0:_F]


def kernel(x, i):
    seg = i.astype(jnp.int32)
    seg3 = seg.reshape(_NB, 1, _B)

    out = pl.pallas_call(
        _fused_kernel,
        grid=(2 * _NB,),
        in_specs=[
            pl.BlockSpec((_B, _F), lambda g: (g % _NB, 0)),
            pl.BlockSpec((1, 1, _B), lambda g: (g % _NB, 0, 0)),
        ],
        out_specs=pl.BlockSpec(
            (_B, _F), lambda g: (jnp.maximum(g - _NB, 0), 0)),
        out_shape=jax.ShapeDtypeStruct((_N, _F), jnp.float32),
        scratch_shapes=[
            pltpu.VMEM((_SPAD, _F), jnp.float32),
            pltpu.VMEM((_SPAD, _F), jnp.float32),
            pltpu.VMEM((_SPAD, 1), jnp.float32),
            pltpu.VMEM((_SPAD, 2 * _F), jnp.bfloat16),
        ],
    )(x, seg3)

    return out


# speedup vs baseline: 3.2322x; 1.0444x over previous
import jax
import jax.numpy as jnp
from jax import lax
from jax.experimental import pallas as pl
from jax.experimental.pallas import tpu as pltpu

_N = 320000
_F = 128
_S = 512
_EPS = 0.001
_B = 10000
_NB = _N // _B
_W = 32
_SPAD = _S + 2 * _W


def _accumulate(oh_bf, oh_f32, x_bf, s0, w, acc_ss, acc_cnt):
    z = jnp.concatenate([x_bf, x_bf * x_bf], axis=1)
    ps = lax.dot_general(oh_bf, z, (((0,), (0,)), ((), ())),
                         preferred_element_type=jnp.float32)
    cnt = jnp.sum(oh_f32, axis=0).reshape(w, 1)
    acc_ss[pl.ds(s0, w), :] += ps
    acc_cnt[pl.ds(s0, w), :] += cnt


def _fused_kernel(x_ref, seg_ref, out_ref, acc_ss, acc_cnt, table):
    g = pl.program_id(0)

    @pl.when(g == 0)
    def _init():
        acc_ss[...] = jnp.zeros_like(acc_ss)
        acc_cnt[...] = jnp.zeros_like(acc_cnt)

    seg = seg_ref[0, 0, :]
    smax = seg[_B - 1]

    @pl.when(g < _NB)
    def _stats_phase():
        s0 = (seg[0] // 8) * 8
        narrow = (smax - s0) < _W

        @pl.when(narrow)
        def _narrow():
            col = lax.broadcasted_iota(jnp.int32, (_B, _W), 1)
            oh_f32 = (seg[:, None] - s0 == col).astype(jnp.float32)
            _accumulate(oh_f32.astype(jnp.bfloat16), oh_f32,
                        x_ref[...].astype(jnp.bfloat16),
                        s0, _W, acc_ss, acc_cnt)

        @pl.when(jnp.logical_not(narrow))
        def _wide():
            col = lax.broadcasted_iota(jnp.int32, (_B, _S), 1)
            oh_f32 = (seg[:, None] == col).astype(jnp.float32)
            _accumulate(oh_f32.astype(jnp.bfloat16), oh_f32,
                        x_ref[...].astype(jnp.bfloat16),
                        0, _S, acc_ss, acc_cnt)

        @pl.when(g == _NB - 1)
        def _finalize():
            cnt = acc_cnt[...]
            mean = acc_ss[:, 0:_F] / jnp.maximum(cnt, 1.0)
            ssq = (jnp.sum(acc_ss[:, _F:2 * _F], axis=1, keepdims=True)
                   - cnt * jnp.sum(mean * mean, axis=1, keepdims=True))
            var = ssq / (cnt * jnp.float32(_F) - 1.0)
            invstd = lax.rsqrt(var + _EPS)
            table[:, 0:_F] = (mean * invstd).astype(jnp.bfloat16)
            table[:, _F:2 * _F] = jnp.broadcast_to(
                invstd, (_SPAD, _F)).astype(jnp.bfloat16)

    @pl.when(g >= _NB)
    def _norm_phase():
        s0 = (seg[0] // 16) * 16
        x = x_ref[...]
        narrow = (smax - s0) < _W

        @pl.when(narrow)
        def _narrow():
            win = table[pl.ds(s0, _W), :]
            col = lax.broadcasted_iota(jnp.int32, (_B, _W), 1)
            oh = (seg[:, None] - s0 == col).astype(
                jnp.float32).astype(jnp.bfloat16)
            rows = lax.dot_general(
                oh, win, (((1,), (0,)), ((), ())),
                preferred_element_type=jnp.float32)
            out_ref[...] = x * rows[:, _F:2 * _F] - rows[:, 0:_F]

        @pl.when(jnp.logical_not(narrow))
        def _wide():
            win = table[pl.ds(0, _S), :]
            col = lax.broadcasted_iota(jnp.int32, (_B, _S), 1)
            oh = (seg[:, None] == col).astype(
                jnp.float32).astype(jnp.bfloat16)
            rows = lax.dot_general(
                oh, win, (((1,), (0,)), ((), ())),
                preferred_element_type=jnp.float32)
            out_ref[...] = x * rows[:, _F:2 * _F] - rows[:, 0:_F]


def kernel(x, i):
    seg = i.astype(jnp.int32)
    seg3 = seg.reshape(_NB, 1, _B)

    out = pl.pallas_call(
        _fused_kernel,
        grid=(2 * _NB,),
        in_specs=[
            pl.BlockSpec((_B, _F), lambda g: (g % _NB, 0)),
            pl.BlockSpec((1, 1, _B), lambda g: (g % _NB, 0, 0)),
        ],
        out_specs=pl.BlockSpec(
            (_B, _F), lambda g: (jnp.maximum(g - _NB, 0), 0)),
        out_shape=jax.ShapeDtypeStruct((_N, _F), jnp.float32),
        scratch_shapes=[
            pltpu.VMEM((_SPAD, 2 * _F), jnp.float32),
            pltpu.VMEM((_SPAD, 1), jnp.float32),
            pltpu.VMEM((_SPAD, 2 * _F), jnp.bfloat16),
        ],
    )(x, seg3)

    return out
